# baseline (speedup 1.0000x reference)
import functools

import jax
import jax.numpy as jnp
from jax import lax
from jax.experimental import pallas as pl
from jax.experimental.pallas import tpu as pltpu
from jax.experimental.pallas import tpu_sc as plsc

TOPK = 128
_CAP = 8192


def _key_of(x_f32):
    s = lax.bitcast_convert_type(x_f32, jnp.int32)
    return s ^ ((s >> 31) & jnp.int32(0x7FFFFFFF))


def _val_of(key_i32):
    return lax.bitcast_convert_type(
        key_i32 ^ ((key_i32 >> 31) & jnp.int32(0x7FFFFFFF)), jnp.float32)


def _vsort_d(v):
    return plsc.sort_key_val(v, v, descending=True)[0]


def _sc_stats(scores2d, h):
    rows, n = scores2d.shape
    nv = n // 16
    info = plsc.get_sparse_core_info()
    NC, NS = info.num_cores, info.num_subcores
    NW = NC * NS
    rpw = rows // NW
    mesh = plsc.VectorSubcoreMesh(core_axis_name="c", subcore_axis_name="s")

    rps = rows // NC

    @functools.partial(
        pl.kernel,
        out_type=jax.ShapeDtypeStruct((rows,), jnp.float32),
        mesh=mesh,
        compiler_params=pltpu.CompilerParams(needs_layout_passes=False),
        scratch_types=[
            pltpu.VMEM((n,), jnp.float32),
            pltpu.VMEM((n,), jnp.float32),
            pltpu.VMEM((_CAP + 32,), jnp.int32),
            pltpu.VMEM((4096,), jnp.int32),
            pltpu.VMEM((256,), jnp.int32),
            pltpu.VMEM((160,), jnp.int32),
            pltpu.VMEM((rpw,), jnp.float32),
            pltpu.VMEM((rpw,), jnp.float32),
            pltpu.VMEM((rpw,), jnp.float32),
            pltpu.VMEM((rpw,), jnp.float32),
            pltpu.VMEM_SHARED((rps,), jnp.float32),
            pltpu.VMEM_SHARED((rps,), jnp.float32),
            pltpu.VMEM_SHARED((rps,), jnp.float32),
            pltpu.VMEM((rps,), jnp.float32),
            pltpu.VMEM((rps,), jnp.float32),
            pltpu.VMEM((rps,), jnp.float32),
            pltpu.SemaphoreType.DMA,
            pltpu.SemaphoreType.DMA,
        ],
    )
    def stats_kernel(scores_hbm, thr_hbm,
                     rowa_v, rowb_v, cand_v, hist_v, hist4_v, top_v,
                     m_t, k_t, a_t, thr_t, m_sh, k_sh, a_sh,
                     m_all, k_all, a_all, sema, semb):
        cid = lax.axis_index("c")
        sid = lax.axis_index("s")
        base_row = cid * rps + sid * rpw

        lane = lax.broadcasted_iota(jnp.int32, (16,), 0)
        ones16 = jnp.ones((16,), jnp.int32)
        lane_f = lane.astype(jnp.float32)
        zero16 = jnp.zeros((16,), jnp.int32)

        def zero_hist():
            @plsc.parallel_loop(0, 256, unroll=8)
            def _(i):
                hist_v[pl.ds(i * 16, 16)] = zero16

        def scan_buckets(target):
            def gt(g, acc):
                s = zero16
                for i in range(16):
                    s = s + hist_v[pl.ds(g * 256 + i * 16, 16)]
                return jnp.where(lane == g, jnp.sum(s), acc)

            gtot = lax.fori_loop(0, 16, gt, zero16)
            suf = jnp.flip(plsc.cumsum(jnp.flip(gtot, 0)), 0)
            G = jnp.max(jnp.where(suf >= target, lane, -1))
            above_g = jnp.sum(jnp.where(lane > G, gtot, 0))

            def ft(i, acc):
                t = jnp.sum(hist_v[pl.ds(G * 256 + i * 16, 16)])
                return jnp.where(lane == i, t, acc)

            ftot = lax.fori_loop(0, 16, ft, zero16)
            suf2 = jnp.flip(plsc.cumsum(jnp.flip(ftot, 0)), 0) + above_g
            bi = jnp.max(jnp.where(suf2 >= target, lane, -1))
            b = G * 16 + bi
            above = jnp.sum(jnp.where(lane > bi, ftot, 0)) + above_g
            return b, above

        def buf_hist4_top(n_c):
            for g in range(16):
                hist4_v[pl.ds(g * 16, 16)] = zero16
            nv_c = (n_c + 15) >> 4

            @plsc.parallel_loop(0, nv_c, unroll=4)
            def _(j):
                kv = cand_v[pl.ds(j * 16, 16)]
                valid = (j * 16 + lane) < n_c
                slot = ((kv >> 28) + 8) * 16 + lane
                plsc.addupdate_scatter(hist4_v, [slot], ones16, mask=valid)

        def buf_hist4(n_c, shp, prefix_check):
            for g in range(16):
                hist4_v[pl.ds(g * 16, 16)] = zero16
            nv_c = (n_c + 15) >> 4

            @plsc.parallel_loop(0, nv_c, unroll=4)
            def _(j):
                kv = cand_v[pl.ds(j * 16, 16)]
                valid = ((j * 16 + lane) < n_c) & \
                    ((kv >> (shp + 4)) == prefix_check)
                slot = ((kv >> shp) & 15) * 16 + lane
                plsc.addupdate_scatter(hist4_v, [slot], ones16, mask=valid)

        def scan16(target):
            ftot = zero16
            for i in range(16):
                t = jnp.sum(hist4_v[pl.ds(i * 16, 16)])
                ftot = jnp.where(lane == i, t, ftot)
            suf = jnp.flip(plsc.cumsum(jnp.flip(ftot, 0)), 0)
            bi = jnp.max(jnp.where(suf >= target, lane, -1))
            above = jnp.sum(jnp.where(lane > bi, ftot, 0))
            return bi, above

        def process(row_v, rr, spec):
            @plsc.parallel_loop(
                0, nv, unroll=16,
                carry=(zero16, jnp.full((16,), -jnp.inf, jnp.float32)))
            def sw1(i, st):
                off, macc = st
                x = row_v[pl.ds(i * 16, 16)]
                key = _key_of(x)
                mask = key >= spec
                pos = off + plsc.cumsum(mask.astype(jnp.int32)) - 1
                plsc.store_scatter(cand_v, [pos], key, mask=mask)
                cnt = plsc.all_reduce_population_count(mask)
                off = jnp.minimum(off + cnt,
                                  jnp.full((16,), _CAP + 1, jnp.int32))
                return off, jnp.maximum(macc, x)

            off, macc = sw1
            n_spec = jnp.max(off)
            m = jnp.max(macc)
            ok = (n_spec >= TOPK) & (n_spec <= _CAP)

            def spec_path():
                buf_hist4_top(n_spec)
                b1, c1 = scan16(jnp.int32(TOPK))
                t1 = (b1 - 8) << 28
                buf_hist4(n_spec, 24, t1 >> 28)
                b2, c2 = scan16(TOPK - c1)
                return t1 | (b2 << 24), c1 + c2, n_spec

            def fallback_path():
                zero_hist()

                @plsc.parallel_loop(0, nv, unroll=8)
                def _(i):
                    x = row_v[pl.ds(i * 16, 16)]
                    key = _key_of(x)
                    slot = (((key >> 24) + 128) * 16) + lane
                    plsc.addupdate_scatter(hist_v, [slot], ones16)

                b0, c_hi = scan_buckets(jnp.int32(TOPK))
                t_lo8 = (b0 - 128) << 24

                @plsc.parallel_loop(0, nv, unroll=8, carry=zero16)
                def swc(i, off2):
                    x = row_v[pl.ds(i * 16, 16)]
                    key = _key_of(x)
                    mask = key >= t_lo8
                    pos = off2 + plsc.cumsum(mask.astype(jnp.int32)) - 1
                    plsc.store_scatter(cand_v, [pos], key, mask=mask)
                    cnt = plsc.all_reduce_population_count(mask)
                    return jnp.minimum(off2 + cnt,
                                       jnp.full((16,), _CAP + 1, jnp.int32))

                return t_lo8, c_hi, jnp.max(swc)

            t_lo8, c_hi, n_c = lax.cond(ok, spec_path, fallback_path)
            spec_next = t_lo8
            t_lo = t_lo8

            for shp in (20, 16, 12, 8, 4, 0):
                buf_hist4(n_c, shp, t_lo >> (shp + 4))
                b, above = scan16(TOPK - c_hi)
                c_hi = c_hi + above
                t_lo = t_lo | (b << shp)
                if shp == 20:
                    nv_c0 = (n_c + 15) >> 4
                    t12 = t_lo

                    @plsc.parallel_loop(0, nv_c0, unroll=4, carry=zero16)
                    def swk(j, off3):
                        kv = cand_v[pl.ds(j * 16, 16)]
                        mask = ((j * 16 + lane) < n_c) & (kv >= t12)
                        pos = off3 + plsc.cumsum(mask.astype(jnp.int32)) - 1
                        plsc.store_scatter(cand_v, [pos], kv, mask=mask)
                        return off3 + plsc.all_reduce_population_count(mask)

                    n_c = jnp.max(swk)

            kstar = t_lo
            ksplat = jnp.full((16,), kstar, jnp.int32)
            for g in range(10):
                top_v[pl.ds(g * 16, 16)] = ksplat

            nv_c = (n_c + 15) >> 4

            @plsc.parallel_loop(0, nv_c, unroll=4, carry=zero16)
            def swt(j, off2):
                kv = cand_v[pl.ds(j * 16, 16)]
                valid = (j * 16 + lane) < n_c
                mask = (kv > kstar) & valid
                pos = off2 + plsc.cumsum(mask.astype(jnp.int32)) - 1
                plsc.store_scatter(top_v, [pos], kv, mask=mask)
                return off2 + plsc.all_reduce_population_count(mask)

            del swt

            w = [_vsort_d(_val_of(top_v[pl.ds(g * 16, 16)]))
                 for g in range(8)]

            def bm32(h):
                p = jnp.maximum(h[0], h[1])
                q = jnp.minimum(h[0], h[1])
                return [_vsort_d(p), _vsort_d(q)]

            def merge2(a, b):
                rb = jnp.flip(b, 0)
                return bm32([jnp.maximum(a, rb), jnp.minimum(a, rb)])

            def merge4(A, B):
                rb = [jnp.flip(B[1], 0), jnp.flip(B[0], 0)]
                hi = [jnp.maximum(A[i], rb[i]) for i in range(2)]
                lo = [jnp.minimum(A[i], rb[i]) for i in range(2)]
                return bm32(hi) + bm32(lo)

            def bm64(h):
                p = [jnp.maximum(h[i], h[i + 2]) for i in range(2)]
                q = [jnp.minimum(h[i], h[i + 2]) for i in range(2)]
                return bm32(p) + bm32(q)

            def merge8(A, B):
                rb = [jnp.flip(B[3 - i], 0) for i in range(4)]
                hi = [jnp.maximum(A[i], rb[i]) for i in range(4)]
                lo = [jnp.minimum(A[i], rb[i]) for i in range(4)]
                return bm64(hi) + bm64(lo)

            s01 = merge2(w[0], w[1])
            s23 = merge2(w[2], w[3])
            s45 = merge2(w[4], w[5])
            s67 = merge2(w[6], w[7])
            q0 = merge4(s01, s23)
            q1 = merge4(s45, s67)
            W = merge8(q0, q1)

            carry = jnp.float32(0.0)
            kcnt = jnp.int32(0)
            for g in range(8):
                S = plsc.cumsum(W[g]) + carry
                jv = (lane + (16 * g + 1)).astype(jnp.float32)
                cond2 = (jv * W[g]) > (S - 1.0)
                kcnt = kcnt + jnp.sum(cond2.astype(jnp.int32))
                carry = carry + jnp.sum(W[g])
            A = jnp.sum(W[0] * jnp.maximum(10.0 - lane_f, 0.0))

            g2 = rr >> 4
            sl = rr & 15
            sel = lane == sl
            mv = m_t[pl.ds(g2 * 16, 16)]
            m_t[pl.ds(g2 * 16, 16)] = jnp.where(sel, m, mv)
            kv2 = k_t[pl.ds(g2 * 16, 16)]
            k_t[pl.ds(g2 * 16, 16)] = jnp.where(sel, kcnt.astype(jnp.float32),
                                                kv2)
            av = a_t[pl.ds(g2 * 16, 16)]
            a_t[pl.ds(g2 * 16, 16)] = jnp.where(sel, A, av)
            return spec_next

        pltpu.async_copy(scores_hbm.at[base_row], rowa_v, sema)

        def pair(i, spec):
            pltpu.async_copy(scores_hbm.at[base_row + 2 * i + 1], rowb_v,
                             semb)
            pltpu.make_async_copy(scores_hbm.at[base_row], rowa_v,
                                  sema).wait()
            spec = process(rowa_v, 2 * i, spec)

            @pl.when(2 * i + 2 < rpw)
            def _():
                pltpu.async_copy(scores_hbm.at[base_row + 2 * i + 2], rowa_v,
                                 sema)

            pltpu.make_async_copy(scores_hbm.at[base_row], rowb_v,
                                  semb).wait()
            spec = process(rowb_v, 2 * i + 1, spec)
            return spec

        lax.fori_loop(0, rpw // 2, pair, jnp.int32(0x7FFFFFFF))

        base_l = sid * rpw
        pltpu.sync_copy(m_t, m_sh.at[pl.ds(base_l, rpw)])
        pltpu.sync_copy(k_t, k_sh.at[pl.ds(base_l, rpw)])
        pltpu.sync_copy(a_t, a_sh.at[pl.ds(base_l, rpw)])
        plsc.subcore_barrier()
        pltpu.sync_copy(m_sh, m_all)
        pltpu.sync_copy(k_sh, k_all)
        pltpu.sync_copy(a_sh, a_all)
        hbase = (base_l // h) * h
        for g in range(rpw // 16):
            kvec = k_t[pl.ds(g * 16, 16)]
            mvec = m_t[pl.ds(g * 16, 16)]
            idx = jnp.clip(kvec.astype(jnp.int32) - 1, 0, h - 1)
            gidx = hbase + idx
            Ag = plsc.load_gather(a_all, [gidx])
            mg = plsc.load_gather(m_all, [gidx])
            tau = (Ag - 55.0 * mg - 10.0) / 10.0 / kvec
            thr_t[pl.ds(g * 16, 16)] = mvec + tau
        pltpu.sync_copy(thr_t, thr_hbm.at[pl.ds(base_row, rpw)])

    return stats_kernel(scores2d)


def _ew_body(thr_ref, x_ref, o_ref):
    thr = thr_ref[0, 0, :][:, None]
    o_ref[...] = jnp.maximum(x_ref[...] - thr[None], 0.0)


def _elementwise(scores, thr):
    B, H, N = scores.shape
    CB = 16384
    grid = (B, N // CB)
    thr3 = thr.reshape(B, 1, H)
    return pl.pallas_call(
        _ew_body,
        grid=grid,
        in_specs=[
            pl.BlockSpec((1, 1, H), lambda b, c: (b, 0, 0)),
            pl.BlockSpec((1, H, CB), lambda b, c: (b, 0, c)),
        ],
        out_specs=pl.BlockSpec((1, H, CB), lambda b, c: (b, 0, c)),
        out_shape=jax.ShapeDtypeStruct((B, H, N), scores.dtype),
    )(thr3, scores)


def kernel(scores):
    B, H, N = scores.shape
    scores2d = scores.reshape(B * H, N)
    thr = _sc_stats(scores2d, H)
    return _elementwise(scores, thr.reshape(B, H))


# speedup vs baseline: 1.1555x; 1.1555x over previous
import functools

import jax
import jax.numpy as jnp
from jax import lax
from jax.experimental import pallas as pl
from jax.experimental.pallas import tpu as pltpu
from jax.experimental.pallas import tpu_sc as plsc

TOPK = 128
_CAP = 8192


def _key_of(x_f32):
    s = lax.bitcast_convert_type(x_f32, jnp.int32)
    return s ^ ((s >> 31) & jnp.int32(0x7FFFFFFF))


def _val_of(key_i32):
    return lax.bitcast_convert_type(
        key_i32 ^ ((key_i32 >> 31) & jnp.int32(0x7FFFFFFF)), jnp.float32)


def _vsort_d(v):
    return plsc.sort_key_val(v, v, descending=True)[0]


def _sc_stats(scores2d, h):
    rows, n = scores2d.shape
    nv = n // 16
    info = plsc.get_sparse_core_info()
    NC, NS = info.num_cores, info.num_subcores
    NW = NC * NS
    rpw = rows // NW
    mesh = plsc.VectorSubcoreMesh(core_axis_name="c", subcore_axis_name="s")

    rps = rows // NC

    @functools.partial(
        pl.kernel,
        out_type=jax.ShapeDtypeStruct((rows,), jnp.float32),
        mesh=mesh,
        compiler_params=pltpu.CompilerParams(needs_layout_passes=False),
        scratch_types=[
            pltpu.VMEM((n,), jnp.float32),
            pltpu.VMEM((n,), jnp.float32),
            pltpu.VMEM((_CAP + 32,), jnp.int32),
            pltpu.VMEM((4096,), jnp.int32),
            pltpu.VMEM((256,), jnp.int32),
            pltpu.VMEM((160,), jnp.int32),
            pltpu.VMEM((rpw,), jnp.float32),
            pltpu.VMEM((rpw,), jnp.float32),
            pltpu.VMEM((rpw,), jnp.float32),
            pltpu.VMEM((rpw,), jnp.float32),
            pltpu.VMEM_SHARED((rps,), jnp.float32),
            pltpu.VMEM_SHARED((rps,), jnp.float32),
            pltpu.VMEM_SHARED((rps,), jnp.float32),
            pltpu.VMEM((rps,), jnp.float32),
            pltpu.VMEM((rps,), jnp.float32),
            pltpu.VMEM((rps,), jnp.float32),
            pltpu.SemaphoreType.DMA,
            pltpu.SemaphoreType.DMA,
        ],
    )
    def stats_kernel(scores_hbm, thr_hbm,
                     rowa_v, rowb_v, cand_v, hist_v, hist4_v, top_v,
                     m_t, k_t, a_t, thr_t, m_sh, k_sh, a_sh,
                     m_all, k_all, a_all, sema, semb):
        cid = lax.axis_index("c")
        sid = lax.axis_index("s")
        base_row = cid * rps + sid * rpw

        lane = lax.broadcasted_iota(jnp.int32, (16,), 0)
        ones16 = jnp.ones((16,), jnp.int32)
        lane_f = lane.astype(jnp.float32)
        zero16 = jnp.zeros((16,), jnp.int32)

        def zero_hist():
            @plsc.parallel_loop(0, 256, unroll=8)
            def _(i):
                hist_v[pl.ds(i * 16, 16)] = zero16

        def scan_buckets(target):
            def gt(g, acc):
                s = zero16
                for i in range(16):
                    s = s + hist_v[pl.ds(g * 256 + i * 16, 16)]
                return jnp.where(lane == g, jnp.sum(s), acc)

            gtot = lax.fori_loop(0, 16, gt, zero16)
            suf = jnp.flip(plsc.cumsum(jnp.flip(gtot, 0)), 0)
            G = jnp.max(jnp.where(suf >= target, lane, -1))
            above_g = jnp.sum(jnp.where(lane > G, gtot, 0))

            def ft(i, acc):
                t = jnp.sum(hist_v[pl.ds(G * 256 + i * 16, 16)])
                return jnp.where(lane == i, t, acc)

            ftot = lax.fori_loop(0, 16, ft, zero16)
            suf2 = jnp.flip(plsc.cumsum(jnp.flip(ftot, 0)), 0) + above_g
            bi = jnp.max(jnp.where(suf2 >= target, lane, -1))
            b = G * 16 + bi
            above = jnp.sum(jnp.where(lane > bi, ftot, 0)) + above_g
            return b, above

        def buf_hist4_top(n_c):
            for g in range(16):
                hist4_v[pl.ds(g * 16, 16)] = zero16
            nv_c = (n_c + 15) >> 4

            @plsc.parallel_loop(0, nv_c, unroll=4)
            def _(j):
                kv = cand_v[pl.ds(j * 16, 16)]
                valid = (j * 16 + lane) < n_c
                slot = ((kv >> 28) + 8) * 16 + lane
                plsc.addupdate_scatter(hist4_v, [slot], ones16, mask=valid)

        def buf_hist4(n_c, shp, prefix_check):
            for g in range(16):
                hist4_v[pl.ds(g * 16, 16)] = zero16
            nv_c = (n_c + 15) >> 4

            @plsc.parallel_loop(0, nv_c, unroll=4)
            def _(j):
                kv = cand_v[pl.ds(j * 16, 16)]
                valid = ((j * 16 + lane) < n_c) & \
                    ((kv >> (shp + 4)) == prefix_check)
                slot = ((kv >> shp) & 15) * 16 + lane
                plsc.addupdate_scatter(hist4_v, [slot], ones16, mask=valid)

        def scan16(target):
            ftot = zero16
            for i in range(16):
                t = jnp.sum(hist4_v[pl.ds(i * 16, 16)])
                ftot = jnp.where(lane == i, t, ftot)
            suf = jnp.flip(plsc.cumsum(jnp.flip(ftot, 0)), 0)
            bi = jnp.max(jnp.where(suf >= target, lane, -1))
            above = jnp.sum(jnp.where(lane > bi, ftot, 0))
            return bi, above

        def process(row_v, rr, spec):
            @plsc.parallel_loop(
                0, nv, unroll=8,
                carry=(zero16, jnp.full((16,), -jnp.inf, jnp.float32)))
            def sw1(i, st):
                off, macc = st
                x = row_v[pl.ds(i * 16, 16)]
                key = _key_of(x)
                mask = key >= spec
                pos = off + plsc.cumsum(mask.astype(jnp.int32)) - 1
                plsc.store_scatter(cand_v, [pos], key, mask=mask)
                cnt = plsc.all_reduce_population_count(mask)
                off = jnp.minimum(off + cnt,
                                  jnp.full((16,), _CAP + 1, jnp.int32))
                return off, jnp.maximum(macc, x)

            off, macc = sw1
            n_spec = jnp.max(off)
            m = jnp.max(macc)
            ok = (n_spec >= TOPK) & (n_spec <= _CAP)

            def spec_path():
                buf_hist4_top(n_spec)
                b1, c1 = scan16(jnp.int32(TOPK))
                t1 = (b1 - 8) << 28
                buf_hist4(n_spec, 24, t1 >> 28)
                b2, c2 = scan16(TOPK - c1)
                return t1 | (b2 << 24), c1 + c2, n_spec

            def fallback_path():
                zero_hist()

                @plsc.parallel_loop(0, nv, unroll=8)
                def _(i):
                    x = row_v[pl.ds(i * 16, 16)]
                    key = _key_of(x)
                    slot = (((key >> 24) + 128) * 16) + lane
                    plsc.addupdate_scatter(hist_v, [slot], ones16)

                b0, c_hi = scan_buckets(jnp.int32(TOPK))
                t_lo8 = (b0 - 128) << 24

                @plsc.parallel_loop(0, nv, unroll=8, carry=zero16)
                def swc(i, off2):
                    x = row_v[pl.ds(i * 16, 16)]
                    key = _key_of(x)
                    mask = key >= t_lo8
                    pos = off2 + plsc.cumsum(mask.astype(jnp.int32)) - 1
                    plsc.store_scatter(cand_v, [pos], key, mask=mask)
                    cnt = plsc.all_reduce_population_count(mask)
                    return jnp.minimum(off2 + cnt,
                                       jnp.full((16,), _CAP + 1, jnp.int32))

                return t_lo8, c_hi, jnp.max(swc)

            t_lo8, c_hi, n_c = lax.cond(ok, spec_path, fallback_path)
            spec_next = t_lo8
            t_lo = t_lo8

            for shp in (20, 16, 12, 8, 4, 0):
                buf_hist4(n_c, shp, t_lo >> (shp + 4))
                b, above = scan16(TOPK - c_hi)
                c_hi = c_hi + above
                t_lo = t_lo | (b << shp)
                if shp == 20:
                    nv_c0 = (n_c + 15) >> 4
                    t12 = t_lo

                    @plsc.parallel_loop(0, nv_c0, unroll=4, carry=zero16)
                    def swk(j, off3):
                        kv = cand_v[pl.ds(j * 16, 16)]
                        mask = ((j * 16 + lane) < n_c) & (kv >= t12)
                        pos = off3 + plsc.cumsum(mask.astype(jnp.int32)) - 1
                        plsc.store_scatter(cand_v, [pos], kv, mask=mask)
                        return off3 + plsc.all_reduce_population_count(mask)

                    n_c = jnp.max(swk)

            kstar = t_lo
            ksplat = jnp.full((16,), kstar, jnp.int32)
            for g in range(10):
                top_v[pl.ds(g * 16, 16)] = ksplat

            nv_c = (n_c + 15) >> 4

            @plsc.parallel_loop(0, nv_c, unroll=4, carry=zero16)
            def swt(j, off2):
                kv = cand_v[pl.ds(j * 16, 16)]
                valid = (j * 16 + lane) < n_c
                mask = (kv > kstar) & valid
                pos = off2 + plsc.cumsum(mask.astype(jnp.int32)) - 1
                plsc.store_scatter(top_v, [pos], kv, mask=mask)
                return off2 + plsc.all_reduce_population_count(mask)

            del swt

            w = [_vsort_d(_val_of(top_v[pl.ds(g * 16, 16)]))
                 for g in range(8)]

            def bm32(h):
                p = jnp.maximum(h[0], h[1])
                q = jnp.minimum(h[0], h[1])
                return [_vsort_d(p), _vsort_d(q)]

            def merge2(a, b):
                rb = jnp.flip(b, 0)
                return bm32([jnp.maximum(a, rb), jnp.minimum(a, rb)])

            def merge4(A, B):
                rb = [jnp.flip(B[1], 0), jnp.flip(B[0], 0)]
                hi = [jnp.maximum(A[i], rb[i]) for i in range(2)]
                lo = [jnp.minimum(A[i], rb[i]) for i in range(2)]
                return bm32(hi) + bm32(lo)

            def bm64(h):
                p = [jnp.maximum(h[i], h[i + 2]) for i in range(2)]
                q = [jnp.minimum(h[i], h[i + 2]) for i in range(2)]
                return bm32(p) + bm32(q)

            def merge8(A, B):
                rb = [jnp.flip(B[3 - i], 0) for i in range(4)]
                hi = [jnp.maximum(A[i], rb[i]) for i in range(4)]
                lo = [jnp.minimum(A[i], rb[i]) for i in range(4)]
                return bm64(hi) + bm64(lo)

            s01 = merge2(w[0], w[1])
            s23 = merge2(w[2], w[3])
            s45 = merge2(w[4], w[5])
            s67 = merge2(w[6], w[7])
            q0 = merge4(s01, s23)
            q1 = merge4(s45, s67)
            W = merge8(q0, q1)

            carry = jnp.float32(0.0)
            kcnt = jnp.int32(0)
            for g in range(8):
                S = plsc.cumsum(W[g]) + carry
                jv = (lane + (16 * g + 1)).astype(jnp.float32)
                cond2 = (jv * W[g]) > (S - 1.0)
                kcnt = kcnt + jnp.sum(cond2.astype(jnp.int32))
                carry = carry + jnp.sum(W[g])
            A = jnp.sum(W[0] * jnp.maximum(10.0 - lane_f, 0.0))

            g2 = rr >> 4
            sl = rr & 15
            sel = lane == sl
            mv = m_t[pl.ds(g2 * 16, 16)]
            m_t[pl.ds(g2 * 16, 16)] = jnp.where(sel, m, mv)
            kv2 = k_t[pl.ds(g2 * 16, 16)]
            k_t[pl.ds(g2 * 16, 16)] = jnp.where(sel, kcnt.astype(jnp.float32),
                                                kv2)
            av = a_t[pl.ds(g2 * 16, 16)]
            a_t[pl.ds(g2 * 16, 16)] = jnp.where(sel, A, av)
            return spec_next

        pltpu.async_copy(scores_hbm.at[base_row], rowa_v, sema)

        def pair(i, spec):
            pltpu.async_copy(scores_hbm.at[base_row + 2 * i + 1], rowb_v,
                             semb)
            pltpu.make_async_copy(scores_hbm.at[base_row], rowa_v,
                                  sema).wait()
            spec = process(rowa_v, 2 * i, spec)

            @pl.when(2 * i + 2 < rpw)
            def _():
                pltpu.async_copy(scores_hbm.at[base_row + 2 * i + 2], rowa_v,
                                 sema)

            pltpu.make_async_copy(scores_hbm.at[base_row], rowb_v,
                                  semb).wait()
            spec = process(rowb_v, 2 * i + 1, spec)
            return spec

        lax.fori_loop(0, rpw // 2, pair, jnp.int32(0x7FFFFFFF))

        base_l = sid * rpw
        pltpu.sync_copy(m_t, m_sh.at[pl.ds(base_l, rpw)])
        pltpu.sync_copy(k_t, k_sh.at[pl.ds(base_l, rpw)])
        pltpu.sync_copy(a_t, a_sh.at[pl.ds(base_l, rpw)])
        plsc.subcore_barrier()
        pltpu.sync_copy(m_sh, m_all)
        pltpu.sync_copy(k_sh, k_all)
        pltpu.sync_copy(a_sh, a_all)
        hbase = (base_l // h) * h
        for g in range(rpw // 16):
            kvec = k_t[pl.ds(g * 16, 16)]
            mvec = m_t[pl.ds(g * 16, 16)]
            idx = jnp.clip(kvec.astype(jnp.int32) - 1, 0, h - 1)
            gidx = hbase + idx
            Ag = plsc.load_gather(a_all, [gidx])
            mg = plsc.load_gather(m_all, [gidx])
            tau = (Ag - 55.0 * mg - 10.0) / 10.0 / kvec
            thr_t[pl.ds(g * 16, 16)] = mvec + tau
        pltpu.sync_copy(thr_t, thr_hbm.at[pl.ds(base_row, rpw)])

    return stats_kernel(scores2d)


def _ew_body(thr_ref, x_ref, o_ref):
    thr = thr_ref[0, 0, :][:, None]
    o_ref[...] = jnp.maximum(x_ref[...] - thr[None], 0.0)


def _elementwise(scores, thr):
    B, H, N = scores.shape
    CB = 16384
    grid = (B, N // CB)
    thr3 = thr.reshape(B, 1, H)
    return pl.pallas_call(
        _ew_body,
        grid=grid,
        in_specs=[
            pl.BlockSpec((1, 1, H), lambda b, c: (b, 0, 0)),
            pl.BlockSpec((1, H, CB), lambda b, c: (b, 0, c)),
        ],
        out_specs=pl.BlockSpec((1, H, CB), lambda b, c: (b, 0, c)),
        out_shape=jax.ShapeDtypeStruct((B, H, N), scores.dtype),
    )(thr3, scores)


def kernel(scores):
    B, H, N = scores.shape
    scores2d = scores.reshape(B * H, N)
    thr = _sc_stats(scores2d, H)
    return _elementwise(scores, thr.reshape(B, H))


# speedup vs baseline: 1.1779x; 1.0194x over previous
import functools

import jax
import jax.numpy as jnp
from jax import lax
from jax.experimental import pallas as pl
from jax.experimental.pallas import tpu as pltpu
from jax.experimental.pallas import tpu_sc as plsc

TOPK = 128
_CAP = 8192


def _key_of(x_f32):
    s = lax.bitcast_convert_type(x_f32, jnp.int32)
    return s ^ ((s >> 31) & jnp.int32(0x7FFFFFFF))


def _val_of(key_i32):
    return lax.bitcast_convert_type(
        key_i32 ^ ((key_i32 >> 31) & jnp.int32(0x7FFFFFFF)), jnp.float32)


def _vsort_d(v):
    return plsc.sort_key_val(v, v, descending=True)[0]


def _sc_stats(scores2d, h):
    rows, n = scores2d.shape
    nv = n // 16
    info = plsc.get_sparse_core_info()
    NC, NS = info.num_cores, info.num_subcores
    NW = NC * NS
    rpw = rows // NW
    mesh = plsc.VectorSubcoreMesh(core_axis_name="c", subcore_axis_name="s")

    rps = rows // NC

    @functools.partial(
        pl.kernel,
        out_type=jax.ShapeDtypeStruct((rows,), jnp.float32),
        mesh=mesh,
        compiler_params=pltpu.CompilerParams(needs_layout_passes=False),
        scratch_types=[
            pltpu.VMEM((n,), jnp.float32),
            pltpu.VMEM((n,), jnp.float32),
            pltpu.VMEM((_CAP + 32,), jnp.int32),
            pltpu.VMEM((4096,), jnp.int32),
            pltpu.VMEM((256,), jnp.int32),
            pltpu.VMEM((160,), jnp.int32),
            pltpu.VMEM((rpw,), jnp.float32),
            pltpu.VMEM((rpw,), jnp.float32),
            pltpu.VMEM((rpw,), jnp.float32),
            pltpu.VMEM((rpw,), jnp.float32),
            pltpu.VMEM_SHARED((rps,), jnp.float32),
            pltpu.VMEM_SHARED((rps,), jnp.float32),
            pltpu.VMEM_SHARED((rps,), jnp.float32),
            pltpu.VMEM((rps,), jnp.float32),
            pltpu.VMEM((rps,), jnp.float32),
            pltpu.VMEM((rps,), jnp.float32),
            pltpu.SemaphoreType.DMA,
            pltpu.SemaphoreType.DMA,
        ],
    )
    def stats_kernel(scores_hbm, thr_hbm,
                     rowa_v, rowb_v, cand_v, hist_v, hist4_v, top_v,
                     m_t, k_t, a_t, thr_t, m_sh, k_sh, a_sh,
                     m_all, k_all, a_all, sema, semb):
        cid = lax.axis_index("c")
        sid = lax.axis_index("s")
        base_row = cid * rps + sid * rpw

        lane = lax.broadcasted_iota(jnp.int32, (16,), 0)
        ones16 = jnp.ones((16,), jnp.int32)
        lane_f = lane.astype(jnp.float32)
        zero16 = jnp.zeros((16,), jnp.int32)

        def zero_hist():
            @plsc.parallel_loop(0, 256, unroll=8)
            def _(i):
                hist_v[pl.ds(i * 16, 16)] = zero16

        def scan_buckets(target):
            def gt(g, acc):
                s = zero16
                for i in range(16):
                    s = s + hist_v[pl.ds(g * 256 + i * 16, 16)]
                return jnp.where(lane == g, jnp.sum(s), acc)

            gtot = lax.fori_loop(0, 16, gt, zero16)
            suf = jnp.flip(plsc.cumsum(jnp.flip(gtot, 0)), 0)
            G = jnp.max(jnp.where(suf >= target, lane, -1))
            above_g = jnp.sum(jnp.where(lane > G, gtot, 0))

            def ft(i, acc):
                t = jnp.sum(hist_v[pl.ds(G * 256 + i * 16, 16)])
                return jnp.where(lane == i, t, acc)

            ftot = lax.fori_loop(0, 16, ft, zero16)
            suf2 = jnp.flip(plsc.cumsum(jnp.flip(ftot, 0)), 0) + above_g
            bi = jnp.max(jnp.where(suf2 >= target, lane, -1))
            b = G * 16 + bi
            above = jnp.sum(jnp.where(lane > bi, ftot, 0)) + above_g
            return b, above

        def buf_hist4_top(n_c):
            for g in range(16):
                hist4_v[pl.ds(g * 16, 16)] = zero16
            nv_c = (n_c + 15) >> 4

            @plsc.parallel_loop(0, nv_c, unroll=4)
            def _(j):
                kv = cand_v[pl.ds(j * 16, 16)]
                valid = (j * 16 + lane) < n_c
                slot = ((kv >> 28) + 8) * 16 + lane
                plsc.addupdate_scatter(hist4_v, [slot], ones16, mask=valid)

        def buf_hist4(n_c, shp, prefix_check):
            for g in range(16):
                hist4_v[pl.ds(g * 16, 16)] = zero16
            nv_c = (n_c + 15) >> 4

            @plsc.parallel_loop(0, nv_c, unroll=4)
            def _(j):
                kv = cand_v[pl.ds(j * 16, 16)]
                valid = ((j * 16 + lane) < n_c) & \
                    ((kv >> (shp + 4)) == prefix_check)
                slot = ((kv >> shp) & 15) * 16 + lane
                plsc.addupdate_scatter(hist4_v, [slot], ones16, mask=valid)

        def scan16(target):
            ftot = zero16
            for i in range(16):
                t = jnp.sum(hist4_v[pl.ds(i * 16, 16)])
                ftot = jnp.where(lane == i, t, ftot)
            suf = jnp.flip(plsc.cumsum(jnp.flip(ftot, 0)), 0)
            bi = jnp.max(jnp.where(suf >= target, lane, -1))
            above = jnp.sum(jnp.where(lane > bi, ftot, 0))
            return bi, above

        def process(row_v, rr, spec):
            @plsc.parallel_loop(
                0, nv, unroll=12,
                carry=(zero16, jnp.full((16,), -jnp.inf, jnp.float32)))
            def sw1(i, st):
                off, macc = st
                x = row_v[pl.ds(i * 16, 16)]
                key = _key_of(x)
                mask = key >= spec
                pos = off + plsc.cumsum(mask.astype(jnp.int32)) - 1
                plsc.store_scatter(cand_v, [pos], key, mask=mask)
                cnt = plsc.all_reduce_population_count(mask)
                off = jnp.minimum(off + cnt,
                                  jnp.full((16,), _CAP + 1, jnp.int32))
                return off, jnp.maximum(macc, x)

            off, macc = sw1
            n_spec = jnp.max(off)
            m = jnp.max(macc)
            ok = (n_spec >= TOPK) & (n_spec <= _CAP)

            def spec_path():
                buf_hist4_top(n_spec)
                b1, c1 = scan16(jnp.int32(TOPK))
                t1 = (b1 - 8) << 28
                buf_hist4(n_spec, 24, t1 >> 28)
                b2, c2 = scan16(TOPK - c1)
                return t1 | (b2 << 24), c1 + c2, n_spec

            def fallback_path():
                zero_hist()

                @plsc.parallel_loop(0, nv, unroll=8)
                def _(i):
                    x = row_v[pl.ds(i * 16, 16)]
                    key = _key_of(x)
                    slot = (((key >> 24) + 128) * 16) + lane
                    plsc.addupdate_scatter(hist_v, [slot], ones16)

                b0, c_hi = scan_buckets(jnp.int32(TOPK))
                t_lo8 = (b0 - 128) << 24

                @plsc.parallel_loop(0, nv, unroll=8, carry=zero16)
                def swc(i, off2):
                    x = row_v[pl.ds(i * 16, 16)]
                    key = _key_of(x)
                    mask = key >= t_lo8
                    pos = off2 + plsc.cumsum(mask.astype(jnp.int32)) - 1
                    plsc.store_scatter(cand_v, [pos], key, mask=mask)
                    cnt = plsc.all_reduce_population_count(mask)
                    return jnp.minimum(off2 + cnt,
                                       jnp.full((16,), _CAP + 1, jnp.int32))

                return t_lo8, c_hi, jnp.max(swc)

            t_lo8, c_hi, n_c = lax.cond(ok, spec_path, fallback_path)
            spec_next = t_lo8
            t_lo = t_lo8

            for shp in (20, 16, 12, 8, 4, 0):
                buf_hist4(n_c, shp, t_lo >> (shp + 4))
                b, above = scan16(TOPK - c_hi)
                c_hi = c_hi + above
                t_lo = t_lo | (b << shp)
                if shp == 20:
                    nv_c0 = (n_c + 15) >> 4
                    t12 = t_lo

                    @plsc.parallel_loop(0, nv_c0, unroll=4, carry=zero16)
                    def swk(j, off3):
                        kv = cand_v[pl.ds(j * 16, 16)]
                        mask = ((j * 16 + lane) < n_c) & (kv >= t12)
                        pos = off3 + plsc.cumsum(mask.astype(jnp.int32)) - 1
                        plsc.store_scatter(cand_v, [pos], kv, mask=mask)
                        return off3 + plsc.all_reduce_population_count(mask)

                    n_c = jnp.max(swk)

            kstar = t_lo
            ksplat = jnp.full((16,), kstar, jnp.int32)
            for g in range(10):
                top_v[pl.ds(g * 16, 16)] = ksplat

            nv_c = (n_c + 15) >> 4

            @plsc.parallel_loop(0, nv_c, unroll=4, carry=zero16)
            def swt(j, off2):
                kv = cand_v[pl.ds(j * 16, 16)]
                valid = (j * 16 + lane) < n_c
                mask = (kv > kstar) & valid
                pos = off2 + plsc.cumsum(mask.astype(jnp.int32)) - 1
                plsc.store_scatter(top_v, [pos], kv, mask=mask)
                return off2 + plsc.all_reduce_population_count(mask)

            del swt

            w = [_vsort_d(_val_of(top_v[pl.ds(g * 16, 16)]))
                 for g in range(8)]

            def bm32(h):
                p = jnp.maximum(h[0], h[1])
                q = jnp.minimum(h[0], h[1])
                return [_vsort_d(p), _vsort_d(q)]

            def merge2(a, b):
                rb = jnp.flip(b, 0)
                return bm32([jnp.maximum(a, rb), jnp.minimum(a, rb)])

            def merge4(A, B):
                rb = [jnp.flip(B[1], 0), jnp.flip(B[0], 0)]
                hi = [jnp.maximum(A[i], rb[i]) for i in range(2)]
                lo = [jnp.minimum(A[i], rb[i]) for i in range(2)]
                return bm32(hi) + bm32(lo)

            def bm64(h):
                p = [jnp.maximum(h[i], h[i + 2]) for i in range(2)]
                q = [jnp.minimum(h[i], h[i + 2]) for i in range(2)]
                return bm32(p) + bm32(q)

            def merge8(A, B):
                rb = [jnp.flip(B[3 - i], 0) for i in range(4)]
                hi = [jnp.maximum(A[i], rb[i]) for i in range(4)]
                lo = [jnp.minimum(A[i], rb[i]) for i in range(4)]
                return bm64(hi) + bm64(lo)

            s01 = merge2(w[0], w[1])
            s23 = merge2(w[2], w[3])
            s45 = merge2(w[4], w[5])
            s67 = merge2(w[6], w[7])
            q0 = merge4(s01, s23)
            q1 = merge4(s45, s67)
            W = merge8(q0, q1)

            carry = jnp.float32(0.0)
            kcnt = jnp.int32(0)
            for g in range(8):
                S = plsc.cumsum(W[g]) + carry
                jv = (lane + (16 * g + 1)).astype(jnp.float32)
                cond2 = (jv * W[g]) > (S - 1.0)
                kcnt = kcnt + jnp.sum(cond2.astype(jnp.int32))
                carry = carry + jnp.sum(W[g])
            A = jnp.sum(W[0] * jnp.maximum(10.0 - lane_f, 0.0))

            g2 = rr >> 4
            sl = rr & 15
            sel = lane == sl
            mv = m_t[pl.ds(g2 * 16, 16)]
            m_t[pl.ds(g2 * 16, 16)] = jnp.where(sel, m, mv)
            kv2 = k_t[pl.ds(g2 * 16, 16)]
            k_t[pl.ds(g2 * 16, 16)] = jnp.where(sel, kcnt.astype(jnp.float32),
                                                kv2)
            av = a_t[pl.ds(g2 * 16, 16)]
            a_t[pl.ds(g2 * 16, 16)] = jnp.where(sel, A, av)
            return spec_next

        pltpu.async_copy(scores_hbm.at[base_row], rowa_v, sema)

        def pair(i, spec):
            pltpu.async_copy(scores_hbm.at[base_row + 2 * i + 1], rowb_v,
                             semb)
            pltpu.make_async_copy(scores_hbm.at[base_row], rowa_v,
                                  sema).wait()
            spec = process(rowa_v, 2 * i, spec)

            @pl.when(2 * i + 2 < rpw)
            def _():
                pltpu.async_copy(scores_hbm.at[base_row + 2 * i + 2], rowa_v,
                                 sema)

            pltpu.make_async_copy(scores_hbm.at[base_row], rowb_v,
                                  semb).wait()
            spec = process(rowb_v, 2 * i + 1, spec)
            return spec

        lax.fori_loop(0, rpw // 2, pair, jnp.int32(0x7FFFFFFF))

        base_l = sid * rpw
        pltpu.sync_copy(m_t, m_sh.at[pl.ds(base_l, rpw)])
        pltpu.sync_copy(k_t, k_sh.at[pl.ds(base_l, rpw)])
        pltpu.sync_copy(a_t, a_sh.at[pl.ds(base_l, rpw)])
        plsc.subcore_barrier()
        pltpu.sync_copy(m_sh, m_all)
        pltpu.sync_copy(k_sh, k_all)
        pltpu.sync_copy(a_sh, a_all)
        hbase = (base_l // h) * h
        for g in range(rpw // 16):
            kvec = k_t[pl.ds(g * 16, 16)]
            mvec = m_t[pl.ds(g * 16, 16)]
            idx = jnp.clip(kvec.astype(jnp.int32) - 1, 0, h - 1)
            gidx = hbase + idx
            Ag = plsc.load_gather(a_all, [gidx])
            mg = plsc.load_gather(m_all, [gidx])
            tau = (Ag - 55.0 * mg - 10.0) / 10.0 / kvec
            thr_t[pl.ds(g * 16, 16)] = mvec + tau
        pltpu.sync_copy(thr_t, thr_hbm.at[pl.ds(base_row, rpw)])

    return stats_kernel(scores2d)


def _ew_body(thr_ref, x_ref, o_ref):
    thr = thr_ref[0, 0, :][:, None]
    o_ref[...] = jnp.maximum(x_ref[...] - thr[None], 0.0)


def _elementwise(scores, thr):
    B, H, N = scores.shape
    CB = 16384
    grid = (B, N // CB)
    thr3 = thr.reshape(B, 1, H)
    return pl.pallas_call(
        _ew_body,
        grid=grid,
        in_specs=[
            pl.BlockSpec((1, 1, H), lambda b, c: (b, 0, 0)),
            pl.BlockSpec((1, H, CB), lambda b, c: (b, 0, c)),
        ],
        out_specs=pl.BlockSpec((1, H, CB), lambda b, c: (b, 0, c)),
        out_shape=jax.ShapeDtypeStruct((B, H, N), scores.dtype),
    )(thr3, scores)


def kernel(scores):
    B, H, N = scores.shape
    scores2d = scores.reshape(B * H, N)
    thr = _sc_stats(scores2d, H)
    return _elementwise(scores, thr.reshape(B, H))
